# Initial kernel scaffold; baseline (speedup 1.0000x reference)
import jax
import jax.numpy as jnp
from jax.experimental import pallas as pl


def kernel(labels, class_means, class_stds, noise):
    raise NotImplementedError("write your pallas kernel here")


# speedup vs baseline: 1.4225x; 1.4225x over previous
import functools

import jax
import jax.numpy as jnp
from jax import lax
from jax.experimental import pallas as pl
from jax.experimental.pallas import tpu as pltpu
from jax.experimental.pallas import tpu_sc as plsc


@functools.lru_cache(maxsize=None)
def _build_sc_kernel(B, NCLS, D):
    info = plsc.get_sparse_core_info()
    NC, NS, L = info.num_cores, info.num_subcores, info.num_lanes
    NW = NC * NS
    BPW = B // NW
    R = 8
    NCHUNK = BPW // R
    U = 8
    GROUPS = D // L

    mesh = plsc.VectorSubcoreMesh(core_axis_name="c", subcore_axis_name="s")

    @functools.partial(
        pl.kernel,
        mesh=mesh,
        out_type=jax.ShapeDtypeStruct((B, D), jnp.float32),
        scratch_types=[
            pltpu.VMEM((BPW,), jnp.int32),
            pltpu.VMEM((R, D), jnp.float32),
            pltpu.VMEM((R, D), jnp.float32),
            pltpu.VMEM((R, D), jnp.float32),
            pltpu.SemaphoreType.DMA,
        ],
    )
    def sc_fma(lab_hbm, mean_hbm, std_hbm, noise_hbm, out_hbm,
               idx_v, mbuf, sbuf, nbuf, sem):
        wid = lax.axis_index("s") * NC + lax.axis_index("c")
        base = wid * BPW
        pltpu.sync_copy(lab_hbm.at[pl.ds(base, BPW)], idx_v)

        def chunk_body(c, _):
            row0 = pl.multiple_of(base + c * R, 8)
            idx = idx_v.at[pl.ds(pl.multiple_of(c * R, 8), R)]
            h_m = pltpu.async_copy(mean_hbm.at[idx], mbuf, sem)
            h_s = pltpu.async_copy(std_hbm.at[idx], sbuf, sem)
            h_n = pltpu.async_copy(noise_hbm.at[pl.ds(row0, R)], nbuf, sem)
            h_m.wait()
            h_s.wait()
            h_n.wait()
            for r in range(R):
                def col_body(i, _, r=r):
                    for u in range(U):
                        off = (i * U + u) * L
                        n = nbuf[r, pl.ds(off, L)]
                        m = mbuf[r, pl.ds(off, L)]
                        s = sbuf[r, pl.ds(off, L)]
                        nbuf[r, pl.ds(off, L)] = m + s * n
                    return 0
                lax.fori_loop(0, GROUPS // U, col_body, 0)
            pltpu.sync_copy(nbuf, out_hbm.at[pl.ds(row0, R)])
            return 0

        lax.fori_loop(0, NCHUNK, chunk_body, 0)

    return sc_fma


def kernel(labels, class_means, class_stds, noise):
    B = labels.shape[0]
    NCLS = class_means.shape[0]
    D = class_means.shape[1] * class_means.shape[2] * class_means.shape[3]
    sc_fma = _build_sc_kernel(B, NCLS, D)
    out = sc_fma(
        labels.astype(jnp.int32),
        class_means.reshape(NCLS, D),
        class_stds.reshape(NCLS, D),
        noise.reshape(B, D),
    )
    return out.reshape(noise.shape)


# speedup vs baseline: 1.4243x; 1.0013x over previous
import functools

import jax
import jax.numpy as jnp
from jax import lax
from jax.experimental import pallas as pl
from jax.experimental.pallas import tpu as pltpu
from jax.experimental.pallas import tpu_sc as plsc


@functools.lru_cache(maxsize=None)
def _build_sc_kernel(B, NCLS, D):
    info = plsc.get_sparse_core_info()
    NC, NS, L = info.num_cores, info.num_subcores, info.num_lanes
    NW = NC * NS
    BPW = B // NW
    R = 4
    NCHUNK = BPW // R
    U = 8
    GROUPS = D // L

    mesh = plsc.VectorSubcoreMesh(core_axis_name="c", subcore_axis_name="s")

    @functools.partial(
        pl.kernel,
        mesh=mesh,
        out_type=jax.ShapeDtypeStruct((B, D), jnp.float32),
        scratch_types=[
            pltpu.VMEM((NCHUNK, R), jnp.int32),
            pltpu.VMEM((R, D), jnp.float32),
            pltpu.VMEM((R, D), jnp.float32),
            pltpu.VMEM((R, D), jnp.float32),
            pltpu.VMEM((R, D), jnp.float32),
            pltpu.VMEM((R, D), jnp.float32),
            pltpu.VMEM((R, D), jnp.float32),
            pltpu.SemaphoreType.DMA,
            pltpu.SemaphoreType.DMA,
            pltpu.SemaphoreType.DMA,
            pltpu.SemaphoreType.DMA,
        ],
    )
    def sc_fma(lab_hbm, mean_hbm, std_hbm, noise_hbm, out_hbm,
               idx_v, mbuf0, mbuf1, sbuf0, sbuf1, nbuf0, nbuf1,
               in_sem0, in_sem1, out_sem0, out_sem1):
        wid = lax.axis_index("s") * NC + lax.axis_index("c")
        base = wid * BPW
        mbufs, sbufs, nbufs = (mbuf0, mbuf1), (sbuf0, sbuf1), (nbuf0, nbuf1)
        in_sems, out_sems = (in_sem0, in_sem1), (out_sem0, out_sem1)

        pltpu.sync_copy(lab_hbm.at[wid], idx_v)

        def issue_in(c, p):
            pltpu.async_copy(mean_hbm.at[idx_v.at[c]], mbufs[p], in_sems[p])
            pltpu.async_copy(std_hbm.at[idx_v.at[c]], sbufs[p], in_sems[p])
            pltpu.async_copy(noise_hbm.at[pl.ds(base + c * R, R)],
                             nbufs[p], in_sems[p])

        def wait_in(p):
            pltpu.make_async_copy(mean_hbm.at[idx_v.at[0]], mbufs[p],
                                  in_sems[p]).wait()
            pltpu.make_async_copy(std_hbm.at[idx_v.at[0]], sbufs[p],
                                  in_sems[p]).wait()
            pltpu.make_async_copy(noise_hbm.at[pl.ds(base, R)], nbufs[p],
                                  in_sems[p]).wait()

        def issue_out(c, p):
            pltpu.async_copy(nbufs[p], out_hbm.at[pl.ds(base + c * R, R)],
                             out_sems[p])

        def wait_out(p):
            pltpu.make_async_copy(nbufs[p], out_hbm.at[pl.ds(base, R)],
                                  out_sems[p]).wait()

        def compute(p):
            mb, sb, nb = mbufs[p], sbufs[p], nbufs[p]
            for r in range(R):
                def col_body(i, _, r=r):
                    for u in range(U):
                        off = (i * U + u) * L
                        n = nb[r, pl.ds(off, L)]
                        m = mb[r, pl.ds(off, L)]
                        s = sb[r, pl.ds(off, L)]
                        nb[r, pl.ds(off, L)] = m + s * n
                    return 0
                lax.fori_loop(0, GROUPS // U, col_body, 0)

        issue_in(0, 0)
        issue_in(1, 1)
        wait_in(0)
        compute(0)
        issue_out(0, 0)

        def pair(i, _):
            for k, p in ((1, 1), (2, 0)):
                c = 2 * i + k
                wait_out(1 - p)
                issue_in(c + 1, 1 - p)
                wait_in(p)
                compute(p)
                issue_out(c, p)
            return 0

        lax.fori_loop(0, (NCHUNK - 2) // 2, pair, 0)

        wait_out(0)
        wait_in(1)
        compute(1)
        issue_out(NCHUNK - 1, 1)
        wait_out(1)

    return sc_fma, NW, NCHUNK, R


def kernel(labels, class_means, class_stds, noise):
    B = labels.shape[0]
    NCLS = class_means.shape[0]
    D = class_means.shape[1] * class_means.shape[2] * class_means.shape[3]
    sc_fma, NW, NCHUNK, R = _build_sc_kernel(B, NCLS, D)
    out = sc_fma(
        labels.astype(jnp.int32).reshape(NW, NCHUNK, R),
        class_means.reshape(NCLS, D),
        class_stds.reshape(NCLS, D),
        noise.reshape(B, D),
    )
    return out.reshape(noise.shape)
